# baseline (speedup 1.0000x reference)
import functools

import jax
import jax.numpy as jnp
from jax import lax
from jax.experimental import pallas as pl
from jax.experimental.pallas import tpu as pltpu, tpu_sc as plsc

N_NODES = 10000
N_EDGES = 160000
HID = 512
NL = 4

NW = 32
E_PAD = NW * 5120
PER_W = E_PAD // NW
A_BLKS = (1024, 1024, 1024, 1024, 1024)

_MESH = plsc.VectorSubcoreMesh(core_axis_name="c", subcore_axis_name="s")
_SC_PARAMS = pltpu.CompilerParams(needs_layout_passes=False,
                                  use_tc_tiling_on_sc=False)


def _silu(v):
    return v * (1.0 / (1.0 + jnp.exp(-v)))


def _ef_body(x_hbm, src_hbm, dst_hbm, ea_hbm, ef_out,
             x_v, src_v, dst_v, ea_v, ef_v):
    c = lax.axis_index("c")
    s = lax.axis_index("s")
    wid = s * 2 + c
    pltpu.sync_copy(x_hbm, x_v)
    iota = lax.iota(jnp.int32, 16)
    base_w = wid * PER_W
    off_b = 0
    for nb in A_BLKS:
        base = base_w + off_b
        pltpu.sync_copy(src_hbm.at[pl.ds(base, nb)], src_v.at[pl.ds(0, nb)])
        pltpu.sync_copy(dst_hbm.at[pl.ds(base, nb)], dst_v.at[pl.ds(0, nb)])
        pltpu.sync_copy(ea_hbm.at[pl.ds(base * 4, nb * 4)],
                        ea_v.at[pl.ds(0, nb * 4)])

        def body(i, carry):
            off = i * 16
            sj = src_v[pl.ds(off, 16)]
            di = dst_v[pl.ds(off, 16)]
            li = off + iota

            def gx(idx, f):
                return plsc.load_gather(x_v, [idx * 5 + f])

            def gea(f):
                return plsc.load_gather(ea_v, [li * 4 + f])

            xi = [gx(di, f) for f in range(5)]
            xj = [gx(sj, f) for f in range(5)]
            u0, u1, u2, rn = gea(0), gea(1), gea(2), gea(3)
            mm = xi[2] * xj[2] + xi[3] * xj[3] + xi[4] * xj[4]
            mr = xj[2] * u0 + xj[3] * u1 + xj[4] * u2
            feats = [xi[2], xi[3], xi[4], xj[2], xj[3], xj[4],
                     mm, mr, rn, xi[0], xi[1], xj[0], xj[1]]
            zero = jnp.zeros((16,), jnp.float32)
            for f in range(16):
                v = feats[f] if f < 13 else zero
                ef_v[f, pl.ds(off, 16)] = v
            return carry

        lax.fori_loop(0, nb // 16, body, 0)
        pltpu.sync_copy(ef_v.at[:, pl.ds(0, nb)], ef_out.at[:, pl.ds(base, nb)])
        off_b += nb


@functools.partial(jax.jit, static_argnums=())
def _build_ef(x, src_p, dst_p, ea_p):
    fn = functools.partial(
        pl.kernel,
        mesh=_MESH,
        compiler_params=_SC_PARAMS,
        out_type=jax.ShapeDtypeStruct((16, E_PAD), jnp.float32),
        scratch_types=[
            pltpu.VMEM((N_NODES * 5,), jnp.float32),
            pltpu.VMEM((1024,), jnp.int32),
            pltpu.VMEM((1024,), jnp.int32),
            pltpu.VMEM((1024 * 4,), jnp.float32),
            pltpu.VMEM((16, 1024), jnp.float32),
        ],
    )(_ef_body)
    return fn(x, src_p, dst_p, ea_p)


def _edge_mlp_body(ef_ref, w1_ref, b1_ref, w2_ref, b2_ref, out_ref):
    efb = ef_ref[...]
    h1 = lax.dot_general(efb, w1_ref[0],
                         dimension_numbers=(((0,), (0,)), ((), ())),
                         preferred_element_type=jnp.float32)
    h1 = _silu(h1 + b1_ref[0])
    msg = jnp.dot(h1, w2_ref[0], preferred_element_type=jnp.float32)
    msg = _silu(msg + b2_ref[0])
    out_ref[...] = msg[None]


def _edge_mlp(ef, We1p, be1r, We2, be2r):
    BE = 2048
    n_e = E_PAD // BE
    return pl.pallas_call(
        _edge_mlp_body,
        grid=(NL, n_e),
        in_specs=[
            pl.BlockSpec((16, BE), lambda l, e: (0, e)),
            pl.BlockSpec((1, 16, HID), lambda l, e: (l, 0, 0)),
            pl.BlockSpec((1, 1, HID), lambda l, e: (l, 0, 0)),
            pl.BlockSpec((1, HID, HID), lambda l, e: (l, 0, 0)),
            pl.BlockSpec((1, 1, HID), lambda l, e: (l, 0, 0)),
        ],
        out_specs=pl.BlockSpec((1, BE, HID), lambda l, e: (l, e, 0)),
        out_shape=jax.ShapeDtypeStruct((NL, E_PAD, HID), jnp.float32),
    )(ef, We1p, be1r, We2, be2r)


E_CHUNK = 512
N_CHUNKS = N_EDGES // E_CHUNK
N_TAIL = N_EDGES - N_CHUNKS * E_CHUNK
N_ROWS_PAD = 10240
RPS = N_ROWS_PAD // 16


def _segsum_body(msg_hbm, dst_hbm, zeros_hbm, aggr,
                 idxb, mbuf, wb, zb, acc):
    c = lax.axis_index("c")
    s = lax.axis_index("s")
    pltpu.sync_copy(zeros_hbm, zb)
    nk = 19 + jnp.where(s < 8, 1, 0)
    start = 19 * s + jnp.minimum(s, 8)
    row0 = RPS * s
    for l in range(NL):
        for j in range(4):
            col0 = (4 * c + j) * 64
            for t in range(5):
                pltpu.sync_copy(zb, acc.at[pl.ds(row0 + 128 * t, 128), :])
            plsc.subcore_barrier()

            def ebody(k, carry):
                kk = start + k
                pltpu.sync_copy(
                    msg_hbm.at[l, pl.ds(kk * E_CHUNK, E_CHUNK), pl.ds(col0, 64)],
                    mbuf)
                for q in range(4):
                    pltpu.sync_copy(
                        dst_hbm.at[pl.ds(kk * E_CHUNK + q * 128, 128)],
                        idxb.at[q])
                    pltpu.sync_copy(mbuf.at[pl.ds(q * 128, 128)],
                                    acc.at[idxb.at[q]], add=True)
                return carry

            lax.fori_loop(0, nk, ebody, 0)

            @pl.when(s == 15)
            def _tail():
                base = N_CHUNKS * E_CHUNK
                pltpu.sync_copy(
                    msg_hbm.at[l, pl.ds(base, N_TAIL), pl.ds(col0, 64)],
                    mbuf.at[pl.ds(0, N_TAIL)])
                for q in range(2):
                    pltpu.sync_copy(dst_hbm.at[pl.ds(base + q * 128, 128)],
                                    idxb.at[q])
                    pltpu.sync_copy(mbuf.at[pl.ds(q * 128, 128)],
                                    acc.at[idxb.at[q]], add=True)

            plsc.subcore_barrier()
            cidx = 4 * c + j
            for t in range(5):
                r = row0 + 128 * t
                pltpu.sync_copy(acc.at[pl.ds(r, 128), :], wb)
                pltpu.sync_copy(wb, aggr.at[l, cidx, pl.ds(r, 128), :])
            plsc.subcore_barrier()


def _segment_sum(msg_hbm, dst_hbm, zeros_hbm):
    fn = functools.partial(
        pl.kernel,
        mesh=_MESH,
        compiler_params=_SC_PARAMS,
        out_type=jax.ShapeDtypeStruct((NL, 8, N_ROWS_PAD, 64), jnp.float32),
        scratch_types=[
            pltpu.VMEM((4, 128), jnp.int32),
            pltpu.VMEM((E_CHUNK, 64), jnp.float32),
            pltpu.VMEM((128, 64), jnp.float32),
            pltpu.VMEM((128, 64), jnp.float32),
            pltpu.VMEM_SHARED((N_ROWS_PAD, 64), jnp.float32),
        ],
    )(_segsum_body)
    return fn(msg_hbm, dst_hbm, zeros_hbm)


def _node_body(x_ref, agg_ref, wemb_ref, bemb_ref, wn1a_ref, wn1b_ref, bn1_ref,
               wn2_ref, bn2_ref, wo1_ref, bo1_ref, wo2_ref, bo2_ref, out_ref):
    xb = x_ref[...]
    h = _silu(jnp.dot(xb, wemb_ref[...], preferred_element_type=jnp.float32)
              + bemb_ref[...])
    for l in range(NL):
        t = (jnp.dot(h, wn1a_ref[l], preferred_element_type=jnp.float32)
             + bn1_ref[l])
        for q in range(8):
            t = t + jnp.dot(agg_ref[l, q], wn1b_ref[l, q],
                            preferred_element_type=jnp.float32)
        upd = jnp.dot(_silu(t), wn2_ref[l], preferred_element_type=jnp.float32)
        h = h + upd + bn2_ref[l]
    o = _silu(jnp.dot(h, wo1_ref[...], preferred_element_type=jnp.float32)
              + bo1_ref[...])
    out_ref[...] = (jnp.dot(o, wo2_ref[...], preferred_element_type=jnp.float32)
                    + bo2_ref[...])


def _node_pipeline(xp, aggr, Wembp, bembr, Wn1a, Wn1b, bn1r, Wn2, bn2r,
                   Wo1, bo1r, Wo2p, bo2r):
    BN = 1000
    n_b = N_NODES // BN
    full = lambda shape: pl.BlockSpec(shape, lambda n: tuple(0 for _ in shape))
    return pl.pallas_call(
        _node_body,
        grid=(n_b,),
        in_specs=[
            pl.BlockSpec((BN, 8), lambda n: (n, 0)),
            pl.BlockSpec((NL, 8, BN, 64), lambda n: (0, 0, n, 0)),
            full((8, HID)),
            full((1, HID)),
            full((NL, HID, HID)),
            full((NL, 8, 64, HID)),
            pl.BlockSpec((NL, 1, HID), lambda n: (0, 0, 0)),
            full((NL, HID, HID)),
            pl.BlockSpec((NL, 1, HID), lambda n: (0, 0, 0)),
            full((HID, HID)),
            full((1, HID)),
            full((HID, 128)),
            full((1, 128)),
        ],
        out_specs=pl.BlockSpec((BN, 128), lambda n: (n, 0)),
        out_shape=jax.ShapeDtypeStruct((N_NODES, 128), jnp.float32),
    )(xp, aggr, Wembp, bembr, Wn1a, Wn1b, bn1r, Wn2, bn2r, Wo1, bo1r, Wo2p, bo2r)


def kernel(x, edge_index, edge_attr, W_emb, b_emb, We1, be1, We2, be2,
           Wn1, bn1, Wn2, bn2, Wo1, bo1, Wo2, bo2):
    ei = edge_index.astype(jnp.int32)
    src = ei[0]
    dst = ei[1]
    pad = E_PAD - N_EDGES
    src_p = jnp.pad(src, (0, pad))
    dst_p = jnp.pad(dst, (0, pad))
    ea_p = jnp.pad(edge_attr, ((0, pad), (0, 0)))

    ef = _build_ef(x.reshape(-1), src_p, dst_p, ea_p.reshape(-1))

    We1p = jnp.pad(We1, ((0, 0), (0, 3), (0, 0)))
    msg = _edge_mlp(ef, We1p, be1.reshape(NL, 1, HID), We2,
                    be2.reshape(NL, 1, HID))

    zeros_hbm = jnp.zeros((128, 64), jnp.float32)
    aggr = _segment_sum(msg, dst, zeros_hbm)

    xp = jnp.pad(x, ((0, 0), (0, 3)))
    Wembp = jnp.pad(W_emb, ((0, 3), (0, 0)))
    Wo2p = jnp.pad(Wo2, ((0, 0), (0, 125)))
    bo2r = jnp.pad(bo2, (0, 125)).reshape(1, 128)
    Wn1a = Wn1[:, :HID, :]
    Wn1b = Wn1[:, HID:, :].reshape(NL, 8, 64, HID)
    out = _node_pipeline(xp, aggr, Wembp, b_emb.reshape(1, HID),
                         Wn1a, Wn1b, bn1.reshape(NL, 1, HID), Wn2,
                         bn2.reshape(NL, 1, HID), Wo1, bo1.reshape(1, HID),
                         Wo2p, bo2r)
    return out[:, :3]


# speedup vs baseline: 1.6021x; 1.6021x over previous
import functools

import jax
import jax.numpy as jnp
from jax import lax
from jax.experimental import pallas as pl
from jax.experimental.pallas import tpu as pltpu, tpu_sc as plsc

N_NODES = 10000
N_EDGES = 160000
HID = 512
NL = 4

NW = 32
E_PAD = NW * 5120
PER_W = E_PAD // NW
A_BLKS = (1024, 1024, 1024, 1024, 1024)

_MESH = plsc.VectorSubcoreMesh(core_axis_name="c", subcore_axis_name="s")
_SC_PARAMS = pltpu.CompilerParams(needs_layout_passes=False,
                                  use_tc_tiling_on_sc=False)


def _silu(v):
    return v * (1.0 / (1.0 + jnp.exp(-v)))


def _ef_body(x_hbm, src_hbm, dst_hbm, ea_hbm, ef_out,
             x_v, src_v, dst_v, ea_v, ef_v):
    c = lax.axis_index("c")
    s = lax.axis_index("s")
    wid = s * 2 + c
    pltpu.sync_copy(x_hbm, x_v)
    iota = lax.iota(jnp.int32, 16)
    base_w = wid * PER_W
    off_b = 0
    for nb in A_BLKS:
        base = base_w + off_b
        pltpu.sync_copy(src_hbm.at[pl.ds(base, nb)], src_v.at[pl.ds(0, nb)])
        pltpu.sync_copy(dst_hbm.at[pl.ds(base, nb)], dst_v.at[pl.ds(0, nb)])
        pltpu.sync_copy(ea_hbm.at[pl.ds(base * 4, nb * 4)],
                        ea_v.at[pl.ds(0, nb * 4)])

        def body(i, carry):
            off = i * 16
            sj = src_v[pl.ds(off, 16)]
            di = dst_v[pl.ds(off, 16)]
            li = off + iota

            def gx(idx, f):
                return plsc.load_gather(x_v, [idx * 5 + f])

            def gea(f):
                return plsc.load_gather(ea_v, [li * 4 + f])

            xi = [gx(di, f) for f in range(5)]
            xj = [gx(sj, f) for f in range(5)]
            u0, u1, u2, rn = gea(0), gea(1), gea(2), gea(3)
            mm = xi[2] * xj[2] + xi[3] * xj[3] + xi[4] * xj[4]
            mr = xj[2] * u0 + xj[3] * u1 + xj[4] * u2
            feats = [xi[2], xi[3], xi[4], xj[2], xj[3], xj[4],
                     mm, mr, rn, xi[0], xi[1], xj[0], xj[1]]
            zero = jnp.zeros((16,), jnp.float32)
            for f in range(16):
                v = feats[f] if f < 13 else zero
                ef_v[f, pl.ds(off, 16)] = v
            return carry

        lax.fori_loop(0, nb // 16, body, 0)
        pltpu.sync_copy(ef_v.at[:, pl.ds(0, nb)], ef_out.at[:, pl.ds(base, nb)])
        off_b += nb


@functools.partial(jax.jit, static_argnums=())
def _build_ef(x, src_p, dst_p, ea_p):
    fn = functools.partial(
        pl.kernel,
        mesh=_MESH,
        compiler_params=_SC_PARAMS,
        out_type=jax.ShapeDtypeStruct((16, E_PAD), jnp.float32),
        scratch_types=[
            pltpu.VMEM((N_NODES * 5,), jnp.float32),
            pltpu.VMEM((1024,), jnp.int32),
            pltpu.VMEM((1024,), jnp.int32),
            pltpu.VMEM((1024 * 4,), jnp.float32),
            pltpu.VMEM((16, 1024), jnp.float32),
        ],
    )(_ef_body)
    return fn(x, src_p, dst_p, ea_p)


def _edge_mlp_body(ef_ref, w1_ref, b1_ref, w2_ref, b2_ref,
                   o0, o1, o2, o3):
    efb = ef_ref[...]
    h1 = lax.dot_general(efb, w1_ref[...],
                         dimension_numbers=(((0,), (0,)), ((), ())),
                         preferred_element_type=jnp.float32)
    h1 = _silu(h1 + b1_ref[...])
    msg = jnp.dot(h1, w2_ref[...], preferred_element_type=jnp.float32)
    msg = _silu(msg + b2_ref[...])
    for a, o in enumerate((o0, o1, o2, o3)):
        o[...] = msg[:, a * 128:(a + 1) * 128]


def _edge_mlp_layer(ef, We1l, be1l, We2l, be2l):
    BE = 2048
    n_e = E_PAD // BE
    ospec = pl.BlockSpec((BE, 128), lambda e: (e, 0))
    oshape = jax.ShapeDtypeStruct((E_PAD, 128), jnp.float32)
    return pl.pallas_call(
        _edge_mlp_body,
        grid=(n_e,),
        in_specs=[
            pl.BlockSpec((16, BE), lambda e: (0, e)),
            pl.BlockSpec((16, HID), lambda e: (0, 0)),
            pl.BlockSpec((1, HID), lambda e: (0, 0)),
            pl.BlockSpec((HID, HID), lambda e: (0, 0)),
            pl.BlockSpec((1, HID), lambda e: (0, 0)),
        ],
        out_specs=(ospec, ospec, ospec, ospec),
        out_shape=(oshape, oshape, oshape, oshape),
    )(ef, We1l, be1l, We2l, be2l)


E_CHUNK = 512
N_CHUNKS = N_EDGES // E_CHUNK
N_TAIL = N_EDGES - N_CHUNKS * E_CHUNK
N_ROWS_PAD = 10240
RPS = N_ROWS_PAD // 16


def _segsum_body(m0, m1, m2, m3, dst_hbm, zeros_hbm, aggr,
                 idxb, mbuf, wb, zb, acc):
    c = lax.axis_index("c")
    s = lax.axis_index("s")
    pltpu.sync_copy(zeros_hbm, zb)
    nk = 19 + jnp.where(s < 8, 1, 0)
    start = 19 * s + jnp.minimum(s, 8)
    row0 = RPS * s
    for a, m_hbm in enumerate((m0, m1, m2, m3)):
        for h in range(2):
            col0 = h * 64
            cidx = 2 * a + h

            @pl.when(c == a // 2)
            def _chunk():
                for t in range(5):
                    pltpu.sync_copy(zb, acc.at[pl.ds(row0 + 128 * t, 128), :])
                plsc.subcore_barrier()

                def ebody(k, carry):
                    kk = start + k
                    pltpu.sync_copy(
                        m_hbm.at[pl.ds(kk * E_CHUNK, E_CHUNK), pl.ds(col0, 64)],
                        mbuf)
                    for q in range(4):
                        pltpu.sync_copy(
                            dst_hbm.at[pl.ds(kk * E_CHUNK + q * 128, 128)],
                            idxb.at[q])
                        pltpu.sync_copy(mbuf.at[pl.ds(q * 128, 128)],
                                        acc.at[idxb.at[q]], add=True)
                    return carry

                lax.fori_loop(0, nk, ebody, 0)

                @pl.when(s == 15)
                def _tail():
                    base = N_CHUNKS * E_CHUNK
                    pltpu.sync_copy(
                        m_hbm.at[pl.ds(base, N_TAIL), pl.ds(col0, 64)],
                        mbuf.at[pl.ds(0, N_TAIL)])
                    for q in range(2):
                        pltpu.sync_copy(dst_hbm.at[pl.ds(base + q * 128, 128)],
                                        idxb.at[q])
                        pltpu.sync_copy(mbuf.at[pl.ds(q * 128, 128)],
                                        acc.at[idxb.at[q]], add=True)

                plsc.subcore_barrier()
                for t in range(5):
                    r = row0 + 128 * t
                    pltpu.sync_copy(acc.at[pl.ds(r, 128), :], wb)
                    pltpu.sync_copy(wb, aggr.at[cidx, pl.ds(r, 128), :])
                plsc.subcore_barrier()


def _segment_sum(msgs, dst_hbm, zeros_hbm):
    fn = functools.partial(
        pl.kernel,
        mesh=_MESH,
        compiler_params=_SC_PARAMS,
        out_type=jax.ShapeDtypeStruct((8, N_ROWS_PAD, 64), jnp.float32),
        scratch_types=[
            pltpu.VMEM((4, 128), jnp.int32),
            pltpu.VMEM((E_CHUNK, 64), jnp.float32),
            pltpu.VMEM((128, 64), jnp.float32),
            pltpu.VMEM((128, 64), jnp.float32),
            pltpu.VMEM_SHARED((N_ROWS_PAD, 64), jnp.float32),
        ],
    )(_segsum_body)
    return fn(msgs[0], msgs[1], msgs[2], msgs[3], dst_hbm, zeros_hbm)


def _node_body(x_ref, ag0, ag1, ag2, ag3, wemb_ref, bemb_ref, wn1a_ref,
               wn1b_ref, bn1_ref, wn2_ref, bn2_ref, wo1_ref, bo1_ref,
               wo2_ref, bo2_ref, out_ref):
    aggs = (ag0, ag1, ag2, ag3)
    xb = x_ref[...]
    h = _silu(jnp.dot(xb, wemb_ref[...], preferred_element_type=jnp.float32)
              + bemb_ref[...])
    for l in range(NL):
        t = (jnp.dot(h, wn1a_ref[l], preferred_element_type=jnp.float32)
             + bn1_ref[l])
        for q in range(8):
            t = t + jnp.dot(aggs[l][q], wn1b_ref[l, q],
                            preferred_element_type=jnp.float32)
        upd = jnp.dot(_silu(t), wn2_ref[l], preferred_element_type=jnp.float32)
        h = h + upd + bn2_ref[l]
    o = _silu(jnp.dot(h, wo1_ref[...], preferred_element_type=jnp.float32)
              + bo1_ref[...])
    out_ref[...] = (jnp.dot(o, wo2_ref[...], preferred_element_type=jnp.float32)
                    + bo2_ref[...])


def _node_pipeline(xp, aggrs, Wembp, bembr, Wn1a, Wn1b, bn1r, Wn2, bn2r,
                   Wo1, bo1r, Wo2p, bo2r):
    BN = 1000
    n_b = N_NODES // BN
    full = lambda shape: pl.BlockSpec(shape, lambda n: tuple(0 for _ in shape))
    agspec = pl.BlockSpec((8, BN, 64), lambda n: (0, n, 0))
    return pl.pallas_call(
        _node_body,
        grid=(n_b,),
        in_specs=[
            pl.BlockSpec((BN, 8), lambda n: (n, 0)),
            agspec, agspec, agspec, agspec,
            full((8, HID)),
            full((1, HID)),
            full((NL, HID, HID)),
            full((NL, 8, 64, HID)),
            pl.BlockSpec((NL, 1, HID), lambda n: (0, 0, 0)),
            full((NL, HID, HID)),
            pl.BlockSpec((NL, 1, HID), lambda n: (0, 0, 0)),
            full((HID, HID)),
            full((1, HID)),
            full((HID, 128)),
            full((1, 128)),
        ],
        out_specs=pl.BlockSpec((BN, 128), lambda n: (n, 0)),
        out_shape=jax.ShapeDtypeStruct((N_NODES, 128), jnp.float32),
    )(xp, aggrs[0], aggrs[1], aggrs[2], aggrs[3], Wembp, bembr, Wn1a, Wn1b,
      bn1r, Wn2, bn2r, Wo1, bo1r, Wo2p, bo2r)


def kernel(x, edge_index, edge_attr, W_emb, b_emb, We1, be1, We2, be2,
           Wn1, bn1, Wn2, bn2, Wo1, bo1, Wo2, bo2):
    ei = edge_index.astype(jnp.int32)
    src = ei[0]
    dst = ei[1]
    pad = E_PAD - N_EDGES
    src_p = jnp.pad(src, (0, pad))
    dst_p = jnp.pad(dst, (0, pad))
    ea_p = jnp.pad(edge_attr, ((0, pad), (0, 0)))

    ef = _build_ef(x.reshape(-1), src_p, dst_p, ea_p.reshape(-1))

    We1p = jnp.pad(We1, ((0, 0), (0, 3), (0, 0)))
    be1r = be1.reshape(NL, 1, HID)
    be2r = be2.reshape(NL, 1, HID)
    zeros_hbm = jnp.zeros((128, 64), jnp.float32)
    aggrs = []
    for l in range(NL):
        msgs = _edge_mlp_layer(ef, We1p[l], be1r[l], We2[l], be2r[l])
        aggrs.append(_segment_sum(msgs, dst, zeros_hbm))

    xp = jnp.pad(x, ((0, 0), (0, 3)))
    Wembp = jnp.pad(W_emb, ((0, 3), (0, 0)))
    Wo2p = jnp.pad(Wo2, ((0, 0), (0, 125)))
    bo2r = jnp.pad(bo2, (0, 125)).reshape(1, 128)
    Wn1a = Wn1[:, :HID, :]
    Wn1b = Wn1[:, HID:, :].reshape(NL, 8, 64, HID)
    out = _node_pipeline(xp, aggrs, Wembp, b_emb.reshape(1, HID),
                         Wn1a, Wn1b, bn1.reshape(NL, 1, HID), Wn2,
                         bn2.reshape(NL, 1, HID), Wo1, bo1.reshape(1, HID),
                         Wo2p, bo2r)
    return out[:, :3]


# speedup vs baseline: 1.6097x; 1.0048x over previous
import functools

import jax
import jax.numpy as jnp
from jax import lax
from jax.experimental import pallas as pl
from jax.experimental.pallas import tpu as pltpu, tpu_sc as plsc

N_NODES = 10000
N_EDGES = 160000
HID = 512
NL = 4

NW = 32
E_PAD = NW * 5120
PER_W = E_PAD // NW
A_BLKS = (1024, 1024, 1024, 1024, 1024)

_MESH = plsc.VectorSubcoreMesh(core_axis_name="c", subcore_axis_name="s")
_SC_PARAMS = pltpu.CompilerParams(needs_layout_passes=False,
                                  use_tc_tiling_on_sc=False)


def _silu(v):
    return v * (1.0 / (1.0 + jnp.exp(-v)))


def _ef_body(x_hbm, src_hbm, dst_hbm, ea_hbm, ef_out,
             x_v, src_v, dst_v, ea_v, ef_v):
    c = lax.axis_index("c")
    s = lax.axis_index("s")
    wid = s * 2 + c
    pltpu.sync_copy(x_hbm, x_v)
    iota = lax.iota(jnp.int32, 16)
    base_w = wid * PER_W
    off_b = 0
    for nb in A_BLKS:
        base = base_w + off_b
        pltpu.sync_copy(src_hbm.at[pl.ds(base, nb)], src_v.at[pl.ds(0, nb)])
        pltpu.sync_copy(dst_hbm.at[pl.ds(base, nb)], dst_v.at[pl.ds(0, nb)])
        pltpu.sync_copy(ea_hbm.at[pl.ds(base * 4, nb * 4)],
                        ea_v.at[pl.ds(0, nb * 4)])

        def body(i, carry):
            off = i * 16
            sj = src_v[pl.ds(off, 16)]
            di = dst_v[pl.ds(off, 16)]
            li = off + iota

            def gx(idx, f):
                return plsc.load_gather(x_v, [idx * 5 + f])

            def gea(f):
                return plsc.load_gather(ea_v, [li * 4 + f])

            xi = [gx(di, f) for f in range(5)]
            xj = [gx(sj, f) for f in range(5)]
            u0, u1, u2, rn = gea(0), gea(1), gea(2), gea(3)
            mm = xi[2] * xj[2] + xi[3] * xj[3] + xi[4] * xj[4]
            mr = xj[2] * u0 + xj[3] * u1 + xj[4] * u2
            feats = [xi[2], xi[3], xi[4], xj[2], xj[3], xj[4],
                     mm, mr, rn, xi[0], xi[1], xj[0], xj[1]]
            zero = jnp.zeros((16,), jnp.float32)
            for f in range(16):
                v = feats[f] if f < 13 else zero
                ef_v[f, pl.ds(off, 16)] = v
            return carry

        lax.fori_loop(0, nb // 16, body, 0)
        pltpu.sync_copy(ef_v.at[:, pl.ds(0, nb)], ef_out.at[:, pl.ds(base, nb)])
        off_b += nb


@functools.partial(jax.jit, static_argnums=())
def _build_ef(x, src_p, dst_p, ea_p):
    fn = functools.partial(
        pl.kernel,
        mesh=_MESH,
        compiler_params=_SC_PARAMS,
        out_type=jax.ShapeDtypeStruct((16, E_PAD), jnp.float32),
        scratch_types=[
            pltpu.VMEM((N_NODES * 5,), jnp.float32),
            pltpu.VMEM((1024,), jnp.int32),
            pltpu.VMEM((1024,), jnp.int32),
            pltpu.VMEM((1024 * 4,), jnp.float32),
            pltpu.VMEM((16, 1024), jnp.float32),
        ],
    )(_ef_body)
    return fn(x, src_p, dst_p, ea_p)


def _edge_mlp_body(ef_ref, w1_ref, b1_ref, w2_ref, b2_ref,
                   o0, o1, o2, o3):
    efb = ef_ref[...]
    h1 = lax.dot_general(efb, w1_ref[...],
                         dimension_numbers=(((0,), (0,)), ((), ())),
                         preferred_element_type=jnp.float32)
    h1 = _silu(h1 + b1_ref[...])
    msg = jnp.dot(h1, w2_ref[...], preferred_element_type=jnp.float32)
    msg = _silu(msg + b2_ref[...])
    for a, o in enumerate((o0, o1, o2, o3)):
        o[...] = msg[:, a * 128:(a + 1) * 128]


def _edge_mlp_layer(ef, We1l, be1l, We2l, be2l):
    BE = 2048
    n_e = E_PAD // BE
    ospec = pl.BlockSpec((BE, 128), lambda e: (e, 0))
    oshape = jax.ShapeDtypeStruct((E_PAD, 128), jnp.float32)
    return pl.pallas_call(
        _edge_mlp_body,
        grid=(n_e,),
        in_specs=[
            pl.BlockSpec((16, BE), lambda e: (0, e)),
            pl.BlockSpec((16, HID), lambda e: (0, 0)),
            pl.BlockSpec((1, HID), lambda e: (0, 0)),
            pl.BlockSpec((HID, HID), lambda e: (0, 0)),
            pl.BlockSpec((1, HID), lambda e: (0, 0)),
        ],
        out_specs=(ospec, ospec, ospec, ospec),
        out_shape=(oshape, oshape, oshape, oshape),
    )(ef, We1l, be1l, We2l, be2l)


E_CHUNK = 512
N_CHUNKS = N_EDGES // E_CHUNK
N_TAIL = N_EDGES - N_CHUNKS * E_CHUNK
N_ROWS_PAD = 10240
RPS = N_ROWS_PAD // 16


def _segsum_body(m0, m1, m2, m3, dst_hbm, zeros_hbm, aggr,
                 idxb, mbuf, zb, acc):
    c = lax.axis_index("c")
    s = lax.axis_index("s")
    pltpu.sync_copy(zeros_hbm, zb)
    nk = 19 + jnp.where(s < 8, 1, 0)
    start = 19 * s + jnp.minimum(s, 8)
    row0 = RPS * s
    def _run_chunk(m_hbm, col0, cidx):
        for t in range(5):
            pltpu.sync_copy(zb, acc.at[pl.ds(row0 + 128 * t, 128), :])
        plsc.subcore_barrier()

        def ebody(k, carry):
            kk = start + k
            pltpu.sync_copy(
                m_hbm.at[pl.ds(kk * E_CHUNK, E_CHUNK), pl.ds(col0, 64)],
                mbuf.at[0])
            for q in range(4):
                pltpu.sync_copy(dst_hbm.at[pl.ds(kk * E_CHUNK + q * 128, 128)],
                                idxb.at[0, q])
            for q in range(4):
                pltpu.sync_copy(mbuf.at[0, pl.ds(q * 128, 128)],
                                acc.at[idxb.at[0, q]], add=True)
            return carry

        lax.fori_loop(0, nk, ebody, 0)

        @pl.when(s == 15)
        def _tail():
            base = N_CHUNKS * E_CHUNK
            pltpu.sync_copy(
                m_hbm.at[pl.ds(base, N_TAIL), pl.ds(col0, 64)],
                mbuf.at[0, pl.ds(0, N_TAIL)])
            for q in range(2):
                pltpu.sync_copy(dst_hbm.at[pl.ds(base + q * 128, 128)],
                                idxb.at[0, q])
                pltpu.sync_copy(mbuf.at[0, pl.ds(q * 128, 128)],
                                acc.at[idxb.at[0, q]], add=True)

        plsc.subcore_barrier()
        for t in range(5):
            r = row0 + 128 * t
            pltpu.sync_copy(acc.at[pl.ds(r, 128), :],
                            aggr.at[cidx, pl.ds(r, 128), :])
        plsc.subcore_barrier()

    for a, m_hbm in enumerate((m0, m1, m2, m3)):
        for h in range(2):
            @pl.when(c == a // 2)
            def _chunk():
                _run_chunk(m_hbm, h * 64, 2 * a + h)


def _segment_sum(msgs, dst_hbm, zeros_hbm):
    fn = functools.partial(
        pl.kernel,
        mesh=_MESH,
        compiler_params=_SC_PARAMS,
        out_type=jax.ShapeDtypeStruct((8, N_ROWS_PAD, 64), jnp.float32),
        scratch_types=[
            pltpu.VMEM((2, 4, 128), jnp.int32),
            pltpu.VMEM((2, E_CHUNK, 64), jnp.float32),
            pltpu.VMEM((128, 64), jnp.float32),
            pltpu.VMEM_SHARED((N_ROWS_PAD, 64), jnp.float32),
        ],
    )(_segsum_body)
    return fn(msgs[0], msgs[1], msgs[2], msgs[3], dst_hbm, zeros_hbm)


def _node_body(x_ref, ag0, ag1, ag2, ag3, wemb_ref, bemb_ref, wn1a_ref,
               wn1b_ref, bn1_ref, wn2_ref, bn2_ref, wo1_ref, bo1_ref,
               wo2_ref, bo2_ref, out_ref):
    aggs = (ag0, ag1, ag2, ag3)
    xb = x_ref[...]
    h = _silu(jnp.dot(xb, wemb_ref[...], preferred_element_type=jnp.float32)
              + bemb_ref[...])
    for l in range(NL):
        t = (jnp.dot(h, wn1a_ref[l], preferred_element_type=jnp.float32)
             + bn1_ref[l])
        for q in range(8):
            t = t + jnp.dot(aggs[l][q], wn1b_ref[l, q],
                            preferred_element_type=jnp.float32)
        upd = jnp.dot(_silu(t), wn2_ref[l], preferred_element_type=jnp.float32)
        h = h + upd + bn2_ref[l]
    o = _silu(jnp.dot(h, wo1_ref[...], preferred_element_type=jnp.float32)
              + bo1_ref[...])
    out_ref[...] = (jnp.dot(o, wo2_ref[...], preferred_element_type=jnp.float32)
                    + bo2_ref[...])


def _node_pipeline(xp, aggrs, Wembp, bembr, Wn1a, Wn1b, bn1r, Wn2, bn2r,
                   Wo1, bo1r, Wo2p, bo2r):
    BN = 1000
    n_b = N_NODES // BN
    full = lambda shape: pl.BlockSpec(shape, lambda n: tuple(0 for _ in shape))
    agspec = pl.BlockSpec((8, BN, 64), lambda n: (0, n, 0))
    return pl.pallas_call(
        _node_body,
        grid=(n_b,),
        in_specs=[
            pl.BlockSpec((BN, 8), lambda n: (n, 0)),
            agspec, agspec, agspec, agspec,
            full((8, HID)),
            full((1, HID)),
            full((NL, HID, HID)),
            full((NL, 8, 64, HID)),
            pl.BlockSpec((NL, 1, HID), lambda n: (0, 0, 0)),
            full((NL, HID, HID)),
            pl.BlockSpec((NL, 1, HID), lambda n: (0, 0, 0)),
            full((HID, HID)),
            full((1, HID)),
            full((HID, 128)),
            full((1, 128)),
        ],
        out_specs=pl.BlockSpec((BN, 128), lambda n: (n, 0)),
        out_shape=jax.ShapeDtypeStruct((N_NODES, 128), jnp.float32),
    )(xp, aggrs[0], aggrs[1], aggrs[2], aggrs[3], Wembp, bembr, Wn1a, Wn1b,
      bn1r, Wn2, bn2r, Wo1, bo1r, Wo2p, bo2r)


def kernel(x, edge_index, edge_attr, W_emb, b_emb, We1, be1, We2, be2,
           Wn1, bn1, Wn2, bn2, Wo1, bo1, Wo2, bo2):
    ei = edge_index.astype(jnp.int32)
    src = ei[0]
    dst = ei[1]
    pad = E_PAD - N_EDGES
    src_p = jnp.pad(src, (0, pad))
    dst_p = jnp.pad(dst, (0, pad))
    ea_p = jnp.pad(edge_attr, ((0, pad), (0, 0)))

    ef = _build_ef(x.reshape(-1), src_p, dst_p, ea_p.reshape(-1))

    We1p = jnp.pad(We1, ((0, 0), (0, 3), (0, 0)))
    be1r = be1.reshape(NL, 1, HID)
    be2r = be2.reshape(NL, 1, HID)
    zeros_hbm = jnp.zeros((128, 64), jnp.float32)
    aggrs = []
    for l in range(NL):
        msgs = _edge_mlp_layer(ef, We1p[l], be1r[l], We2[l], be2r[l])
        aggrs.append(_segment_sum(msgs, dst, zeros_hbm))

    xp = jnp.pad(x, ((0, 0), (0, 3)))
    Wembp = jnp.pad(W_emb, ((0, 3), (0, 0)))
    Wo2p = jnp.pad(Wo2, ((0, 0), (0, 125)))
    bo2r = jnp.pad(bo2, (0, 125)).reshape(1, 128)
    Wn1a = Wn1[:, :HID, :]
    Wn1b = Wn1[:, HID:, :].reshape(NL, 8, 64, HID)
    out = _node_pipeline(xp, aggrs, Wembp, b_emb.reshape(1, HID),
                         Wn1a, Wn1b, bn1.reshape(NL, 1, HID), Wn2,
                         bn2.reshape(NL, 1, HID), Wo1, bo1.reshape(1, HID),
                         Wo2p, bo2r)
    return out[:, :3]


# speedup vs baseline: 1.9249x; 1.1958x over previous
import functools

import jax
import jax.numpy as jnp
from jax import lax
from jax.experimental import pallas as pl
from jax.experimental.pallas import tpu as pltpu, tpu_sc as plsc

N_NODES = 10000
N_EDGES = 160000
HID = 512
NL = 4

NW = 32
E_PAD = NW * 5120
PER_W = E_PAD // NW
A_BLKS = (1024, 1024, 1024, 1024, 1024)

_MESH = plsc.VectorSubcoreMesh(core_axis_name="c", subcore_axis_name="s")
_SC_PARAMS = pltpu.CompilerParams(needs_layout_passes=False,
                                  use_tc_tiling_on_sc=False)


def _silu(v):
    return v * (1.0 / (1.0 + jnp.exp(-v)))


def _ef_body(x_hbm, src_hbm, dst_hbm, ea_hbm, ef_out,
             x_v, src_v, dst_v, ea_v, ef_v):
    c = lax.axis_index("c")
    s = lax.axis_index("s")
    wid = s * 2 + c
    pltpu.sync_copy(x_hbm, x_v)
    iota = lax.iota(jnp.int32, 16)
    base_w = wid * PER_W
    off_b = 0
    for nb in A_BLKS:
        base = base_w + off_b
        pltpu.sync_copy(src_hbm.at[pl.ds(base, nb)], src_v.at[pl.ds(0, nb)])
        pltpu.sync_copy(dst_hbm.at[pl.ds(base, nb)], dst_v.at[pl.ds(0, nb)])
        pltpu.sync_copy(ea_hbm.at[pl.ds(base * 4, nb * 4)],
                        ea_v.at[pl.ds(0, nb * 4)])

        def body(i, carry):
            off = i * 16
            sj = src_v[pl.ds(off, 16)]
            di = dst_v[pl.ds(off, 16)]
            li = off + iota

            def gx(idx, f):
                return plsc.load_gather(x_v, [idx * 5 + f])

            def gea(f):
                return plsc.load_gather(ea_v, [li * 4 + f])

            xi = [gx(di, f) for f in range(5)]
            xj = [gx(sj, f) for f in range(5)]
            u0, u1, u2, rn = gea(0), gea(1), gea(2), gea(3)
            mm = xi[2] * xj[2] + xi[3] * xj[3] + xi[4] * xj[4]
            mr = xj[2] * u0 + xj[3] * u1 + xj[4] * u2
            feats = [xi[2], xi[3], xi[4], xj[2], xj[3], xj[4],
                     mm, mr, rn, xi[0], xi[1], xj[0], xj[1]]
            zero = jnp.zeros((16,), jnp.float32)
            for f in range(16):
                v = feats[f] if f < 13 else zero
                ef_v[f, pl.ds(off, 16)] = v
            return carry

        lax.fori_loop(0, nb // 16, body, 0)
        pltpu.sync_copy(ef_v.at[:, pl.ds(0, nb)], ef_out.at[:, pl.ds(base, nb)])
        off_b += nb


@functools.partial(jax.jit, static_argnums=())
def _build_ef(x, src_p, dst_p, ea_p):
    fn = functools.partial(
        pl.kernel,
        mesh=_MESH,
        compiler_params=_SC_PARAMS,
        out_type=jax.ShapeDtypeStruct((16, E_PAD), jnp.float32),
        scratch_types=[
            pltpu.VMEM((N_NODES * 5,), jnp.float32),
            pltpu.VMEM((1024,), jnp.int32),
            pltpu.VMEM((1024,), jnp.int32),
            pltpu.VMEM((1024 * 4,), jnp.float32),
            pltpu.VMEM((16, 1024), jnp.float32),
        ],
    )(_ef_body)
    return fn(x, src_p, dst_p, ea_p)


def _edge_mlp_body(ef_ref, w1_ref, b1_ref, w2_ref, b2_ref,
                   o0, o1, o2, o3):
    efb = ef_ref[...]
    h1 = lax.dot_general(efb, w1_ref[...],
                         dimension_numbers=(((0,), (0,)), ((), ())),
                         preferred_element_type=jnp.float32)
    h1 = _silu(h1 + b1_ref[...])
    msg = jnp.dot(h1, w2_ref[...], preferred_element_type=jnp.float32)
    msg = _silu(msg + b2_ref[...])
    for a, o in enumerate((o0, o1, o2, o3)):
        o[...] = msg[:, a * 128:(a + 1) * 128]


def _edge_mlp_layer(ef, We1l, be1l, We2l, be2l):
    BE = 2048
    n_e = E_PAD // BE
    ospec = pl.BlockSpec((BE, 128), lambda e: (e, 0))
    oshape = jax.ShapeDtypeStruct((E_PAD, 128), jnp.float32)
    return pl.pallas_call(
        _edge_mlp_body,
        grid=(n_e,),
        in_specs=[
            pl.BlockSpec((16, BE), lambda e: (0, e)),
            pl.BlockSpec((16, HID), lambda e: (0, 0)),
            pl.BlockSpec((1, HID), lambda e: (0, 0)),
            pl.BlockSpec((HID, HID), lambda e: (0, 0)),
            pl.BlockSpec((1, HID), lambda e: (0, 0)),
        ],
        out_specs=(ospec, ospec, ospec, ospec),
        out_shape=(oshape, oshape, oshape, oshape),
    )(ef, We1l, be1l, We2l, be2l)


E_CHUNK = 512
N_CHUNKS = N_EDGES // E_CHUNK
N_TAIL = N_EDGES - N_CHUNKS * E_CHUNK
N_ROWS_PAD = 10240
RPS = N_ROWS_PAD // 16


def _segsum_body(m0, m1, m2, m3, dst_hbm, zeros_hbm, aggr,
                 idxb, mbuf, zb, acc):
    c = lax.axis_index("c")
    s = lax.axis_index("s")
    pltpu.sync_copy(zeros_hbm, zb)
    nk = 19 + jnp.where(s < 8, 1, 0)
    start = 19 * s + jnp.minimum(s, 8)
    row0 = RPS * s
    def _run_chunk(m_hbm, col0, cidx):
        for t in range(5):
            pltpu.sync_copy(zb, acc.at[pl.ds(row0 + 128 * t, 128), :])
        plsc.subcore_barrier()

        def ebody(k, carry):
            kk = start + k
            pltpu.sync_copy(
                m_hbm.at[pl.ds(kk * E_CHUNK, E_CHUNK), pl.ds(col0, 64)],
                mbuf.at[0])
            pltpu.sync_copy(dst_hbm.at[kk], idxb.at[0])
            for q in range(4):
                pltpu.sync_copy(mbuf.at[0, pl.ds(q * 128, 128)],
                                acc.at[idxb.at[0, q]], add=True)
            return carry

        lax.fori_loop(0, nk, ebody, 0)

        @pl.when(s == 15)
        def _tail():
            base = N_CHUNKS * E_CHUNK
            pltpu.sync_copy(
                m_hbm.at[pl.ds(base, N_TAIL), pl.ds(col0, 64)],
                mbuf.at[0, pl.ds(0, N_TAIL)])
            pltpu.sync_copy(dst_hbm.at[N_CHUNKS], idxb.at[0])
            for q in range(2):
                pltpu.sync_copy(mbuf.at[0, pl.ds(q * 128, 128)],
                                acc.at[idxb.at[0, q]], add=True)

        plsc.subcore_barrier()
        for t in range(5):
            r = row0 + 128 * t
            pltpu.sync_copy(acc.at[pl.ds(r, 128), :],
                            aggr.at[cidx, pl.ds(r, 128), :])
        plsc.subcore_barrier()

    for a, m_hbm in enumerate((m0, m1, m2, m3)):
        for h in range(2):
            @pl.when(c == a // 2)
            def _chunk():
                _run_chunk(m_hbm, h * 64, 2 * a + h)


def _segment_sum(msgs, dst_hbm, zeros_hbm):
    fn = functools.partial(
        pl.kernel,
        mesh=_MESH,
        compiler_params=_SC_PARAMS,
        out_type=jax.ShapeDtypeStruct((8, N_ROWS_PAD, 64), jnp.float32),
        scratch_types=[
            pltpu.VMEM((2, 4, 128), jnp.int32),
            pltpu.VMEM((2, E_CHUNK, 64), jnp.float32),
            pltpu.VMEM((128, 64), jnp.float32),
            pltpu.VMEM_SHARED((N_ROWS_PAD, 64), jnp.float32),
        ],
    )(_segsum_body)
    return fn(msgs[0], msgs[1], msgs[2], msgs[3], dst_hbm, zeros_hbm)


def _node_body(x_ref, ag0, ag1, ag2, ag3, wemb_ref, bemb_ref, wn1a_ref,
               wn1b_ref, bn1_ref, wn2_ref, bn2_ref, wo1_ref, bo1_ref,
               wo2_ref, bo2_ref, out_ref):
    aggs = (ag0, ag1, ag2, ag3)
    xb = x_ref[...]
    h = _silu(jnp.dot(xb, wemb_ref[...], preferred_element_type=jnp.float32)
              + bemb_ref[...])
    for l in range(NL):
        t = (jnp.dot(h, wn1a_ref[l], preferred_element_type=jnp.float32)
             + bn1_ref[l])
        for q in range(8):
            t = t + jnp.dot(aggs[l][q], wn1b_ref[l, q],
                            preferred_element_type=jnp.float32)
        upd = jnp.dot(_silu(t), wn2_ref[l], preferred_element_type=jnp.float32)
        h = h + upd + bn2_ref[l]
    o = _silu(jnp.dot(h, wo1_ref[...], preferred_element_type=jnp.float32)
              + bo1_ref[...])
    out_ref[...] = (jnp.dot(o, wo2_ref[...], preferred_element_type=jnp.float32)
                    + bo2_ref[...])


def _node_pipeline(xp, aggrs, Wembp, bembr, Wn1a, Wn1b, bn1r, Wn2, bn2r,
                   Wo1, bo1r, Wo2p, bo2r):
    BN = 1000
    n_b = N_NODES // BN
    full = lambda shape: pl.BlockSpec(shape, lambda n: tuple(0 for _ in shape))
    agspec = pl.BlockSpec((8, BN, 64), lambda n: (0, n, 0))
    return pl.pallas_call(
        _node_body,
        grid=(n_b,),
        in_specs=[
            pl.BlockSpec((BN, 8), lambda n: (n, 0)),
            agspec, agspec, agspec, agspec,
            full((8, HID)),
            full((1, HID)),
            full((NL, HID, HID)),
            full((NL, 8, 64, HID)),
            pl.BlockSpec((NL, 1, HID), lambda n: (0, 0, 0)),
            full((NL, HID, HID)),
            pl.BlockSpec((NL, 1, HID), lambda n: (0, 0, 0)),
            full((HID, HID)),
            full((1, HID)),
            full((HID, 128)),
            full((1, 128)),
        ],
        out_specs=pl.BlockSpec((BN, 128), lambda n: (n, 0)),
        out_shape=jax.ShapeDtypeStruct((N_NODES, 128), jnp.float32),
    )(xp, aggrs[0], aggrs[1], aggrs[2], aggrs[3], Wembp, bembr, Wn1a, Wn1b,
      bn1r, Wn2, bn2r, Wo1, bo1r, Wo2p, bo2r)


def kernel(x, edge_index, edge_attr, W_emb, b_emb, We1, be1, We2, be2,
           Wn1, bn1, Wn2, bn2, Wo1, bo1, Wo2, bo2):
    ei = edge_index.astype(jnp.int32)
    src = ei[0]
    dst = ei[1]
    pad = E_PAD - N_EDGES
    src_p = jnp.pad(src, (0, pad))
    dst_p = jnp.pad(dst, (0, pad))
    ea_p = jnp.pad(edge_attr, ((0, pad), (0, 0)))

    ef = _build_ef(x.reshape(-1), src_p, dst_p, ea_p.reshape(-1))

    We1p = jnp.pad(We1, ((0, 0), (0, 3), (0, 0)))
    be1r = be1.reshape(NL, 1, HID)
    be2r = be2.reshape(NL, 1, HID)
    zeros_hbm = jnp.zeros((128, 64), jnp.float32)
    dst3 = jnp.pad(dst, (0, (N_CHUNKS + 1) * E_CHUNK - N_EDGES)).reshape(
        N_CHUNKS + 1, 4, 128)
    aggrs = []
    for l in range(NL):
        msgs = _edge_mlp_layer(ef, We1p[l], be1r[l], We2[l], be2r[l])
        aggrs.append(_segment_sum(msgs, dst3, zeros_hbm))

    xp = jnp.pad(x, ((0, 0), (0, 3)))
    Wembp = jnp.pad(W_emb, ((0, 3), (0, 0)))
    Wo2p = jnp.pad(Wo2, ((0, 0), (0, 125)))
    bo2r = jnp.pad(bo2, (0, 125)).reshape(1, 128)
    Wn1a = Wn1[:, :HID, :]
    Wn1b = Wn1[:, HID:, :].reshape(NL, 8, 64, HID)
    out = _node_pipeline(xp, aggrs, Wembp, b_emb.reshape(1, HID),
                         Wn1a, Wn1b, bn1.reshape(NL, 1, HID), Wn2,
                         bn2.reshape(NL, 1, HID), Wo1, bo1.reshape(1, HID),
                         Wo2p, bo2r)
    return out[:, :3]


# speedup vs baseline: 1.9695x; 1.0232x over previous
import functools

import jax
import jax.numpy as jnp
from jax import lax
from jax.experimental import pallas as pl
from jax.experimental.pallas import tpu as pltpu, tpu_sc as plsc

N_NODES = 10000
N_EDGES = 160000
HID = 512
NL = 4

NW = 32
E_PAD = NW * 5120
PER_W = E_PAD // NW
A_BLKS = (1024, 1024, 1024, 1024, 1024)

_MESH = plsc.VectorSubcoreMesh(core_axis_name="c", subcore_axis_name="s")
_SC_PARAMS = pltpu.CompilerParams(needs_layout_passes=False,
                                  use_tc_tiling_on_sc=False)


def _silu(v):
    return v * (1.0 / (1.0 + jnp.exp(-v)))


def _ef_body(x_hbm, src_hbm, dst_hbm, ea_hbm, ef_out,
             x_v, src_v, dst_v, ea_v, ef_v):
    c = lax.axis_index("c")
    s = lax.axis_index("s")
    wid = s * 2 + c
    pltpu.sync_copy(x_hbm, x_v)
    iota = lax.iota(jnp.int32, 16)
    base_w = wid * PER_W
    off_b = 0
    for nb in A_BLKS:
        base = base_w + off_b
        pltpu.sync_copy(src_hbm.at[pl.ds(base, nb)], src_v.at[pl.ds(0, nb)])
        pltpu.sync_copy(dst_hbm.at[pl.ds(base, nb)], dst_v.at[pl.ds(0, nb)])
        pltpu.sync_copy(ea_hbm.at[pl.ds(base * 4, nb * 4)],
                        ea_v.at[pl.ds(0, nb * 4)])

        def body(i, carry):
            off = i * 16
            sj = src_v[pl.ds(off, 16)]
            di = dst_v[pl.ds(off, 16)]
            li = off + iota

            def gx(idx, f):
                return plsc.load_gather(x_v, [idx * 5 + f])

            def gea(f):
                return plsc.load_gather(ea_v, [li * 4 + f])

            xi = [gx(di, f) for f in range(5)]
            xj = [gx(sj, f) for f in range(5)]
            u0, u1, u2, rn = gea(0), gea(1), gea(2), gea(3)
            mm = xi[2] * xj[2] + xi[3] * xj[3] + xi[4] * xj[4]
            mr = xj[2] * u0 + xj[3] * u1 + xj[4] * u2
            feats = [xi[2], xi[3], xi[4], xj[2], xj[3], xj[4],
                     mm, mr, rn, xi[0], xi[1], xj[0], xj[1]]
            zero = jnp.zeros((16,), jnp.float32)
            for f in range(16):
                v = feats[f] if f < 13 else zero
                ef_v[f, pl.ds(off, 16)] = v
            return carry

        lax.fori_loop(0, nb // 16, body, 0)
        pltpu.sync_copy(ef_v.at[:, pl.ds(0, nb)], ef_out.at[:, pl.ds(base, nb)])
        off_b += nb


@functools.partial(jax.jit, static_argnums=())
def _build_ef(x, src_p, dst_p, ea_p):
    fn = functools.partial(
        pl.kernel,
        mesh=_MESH,
        compiler_params=_SC_PARAMS,
        out_type=jax.ShapeDtypeStruct((16, E_PAD), jnp.float32),
        scratch_types=[
            pltpu.VMEM((N_NODES * 5,), jnp.float32),
            pltpu.VMEM((1024,), jnp.int32),
            pltpu.VMEM((1024,), jnp.int32),
            pltpu.VMEM((1024 * 4,), jnp.float32),
            pltpu.VMEM((16, 1024), jnp.float32),
        ],
    )(_ef_body)
    return fn(x, src_p, dst_p, ea_p)


def _edge_mlp_body(ef_ref, w1_ref, b1_ref, w2_ref, b2_ref,
                   o0, o1, o2, o3):
    efb = ef_ref[...]
    h1 = lax.dot_general(efb, w1_ref[...],
                         dimension_numbers=(((0,), (0,)), ((), ())),
                         preferred_element_type=jnp.float32)
    h1 = _silu(h1 + b1_ref[...])
    msg = jnp.dot(h1, w2_ref[...], preferred_element_type=jnp.float32)
    msg = _silu(msg + b2_ref[...])
    for a, o in enumerate((o0, o1, o2, o3)):
        o[...] = msg[:, a * 128:(a + 1) * 128]


def _edge_mlp_layer(ef, We1l, be1l, We2l, be2l):
    BE = 2048
    n_e = E_PAD // BE
    ospec = pl.BlockSpec((BE, 128), lambda e: (e, 0))
    oshape = jax.ShapeDtypeStruct((E_PAD, 128), jnp.float32)
    return pl.pallas_call(
        _edge_mlp_body,
        grid=(n_e,),
        in_specs=[
            pl.BlockSpec((16, BE), lambda e: (0, e)),
            pl.BlockSpec((16, HID), lambda e: (0, 0)),
            pl.BlockSpec((1, HID), lambda e: (0, 0)),
            pl.BlockSpec((HID, HID), lambda e: (0, 0)),
            pl.BlockSpec((1, HID), lambda e: (0, 0)),
        ],
        out_specs=(ospec, ospec, ospec, ospec),
        out_shape=(oshape, oshape, oshape, oshape),
    )(ef, We1l, be1l, We2l, be2l)


E_CHUNK = 512
N_CHUNKS = N_EDGES // E_CHUNK
N_TAIL = N_EDGES - N_CHUNKS * E_CHUNK
N_ROWS_PAD = 10240
RPS = N_ROWS_PAD // 16


def _segsum_body(m0, m1, m2, m3, dst_hbm, zeros_hbm, aggr,
                 idxb, idxt, mbuf, zb, acc):
    c = lax.axis_index("c")
    s = lax.axis_index("s")
    pltpu.sync_copy(zeros_hbm, zb)
    nk = 19 + jnp.where(s < 8, 1, 0)
    start = 19 * s + jnp.minimum(s, 8)
    row0 = RPS * s
    def _run_chunk(m_hbm, col0, cidx):
        for t in range(5):
            pltpu.sync_copy(zb, acc.at[pl.ds(row0 + 128 * t, 128), :])
        plsc.subcore_barrier()

        def ebody(k, carry):
            kk = start + k
            pltpu.sync_copy(
                m_hbm.at[pl.ds(kk * E_CHUNK, E_CHUNK), pl.ds(col0, 64)],
                mbuf.at[0])
            pltpu.sync_copy(dst_hbm.at[kk], idxb)
            pltpu.sync_copy(mbuf.at[0], acc.at[idxb], add=True)
            return carry

        lax.fori_loop(0, nk, ebody, 0)

        @pl.when(s == 15)
        def _tail():
            base = N_CHUNKS * E_CHUNK
            pltpu.sync_copy(
                m_hbm.at[pl.ds(base, N_TAIL), pl.ds(col0, 64)],
                mbuf.at[0, pl.ds(0, N_TAIL)])
            pltpu.sync_copy(dst_hbm.at[N_CHUNKS, pl.ds(0, N_TAIL)], idxt)
            pltpu.sync_copy(mbuf.at[0, pl.ds(0, N_TAIL)],
                            acc.at[idxt], add=True)

        plsc.subcore_barrier()
        for t in range(5):
            r = row0 + 128 * t
            pltpu.sync_copy(acc.at[pl.ds(r, 128), :],
                            aggr.at[cidx, pl.ds(r, 128), :])
        plsc.subcore_barrier()

    for a, m_hbm in enumerate((m0, m1, m2, m3)):
        for h in range(2):
            @pl.when(c == a // 2)
            def _chunk():
                _run_chunk(m_hbm, h * 64, 2 * a + h)


def _segment_sum(msgs, dst_hbm, zeros_hbm):
    fn = functools.partial(
        pl.kernel,
        mesh=_MESH,
        compiler_params=_SC_PARAMS,
        out_type=jax.ShapeDtypeStruct((8, N_ROWS_PAD, 64), jnp.float32),
        scratch_types=[
            pltpu.VMEM((E_CHUNK,), jnp.int32),
            pltpu.VMEM((N_TAIL,), jnp.int32),
            pltpu.VMEM((2, E_CHUNK, 64), jnp.float32),
            pltpu.VMEM((128, 64), jnp.float32),
            pltpu.VMEM_SHARED((N_ROWS_PAD, 64), jnp.float32),
        ],
    )(_segsum_body)
    return fn(msgs[0], msgs[1], msgs[2], msgs[3], dst_hbm, zeros_hbm)


def _node_body(x_ref, ag0, ag1, ag2, ag3, wemb_ref, bemb_ref, wn1a_ref,
               wn1b_ref, bn1_ref, wn2_ref, bn2_ref, wo1_ref, bo1_ref,
               wo2_ref, bo2_ref, out_ref):
    aggs = (ag0, ag1, ag2, ag3)
    xb = x_ref[...]
    h = _silu(jnp.dot(xb, wemb_ref[...], preferred_element_type=jnp.float32)
              + bemb_ref[...])
    for l in range(NL):
        t = (jnp.dot(h, wn1a_ref[l], preferred_element_type=jnp.float32)
             + bn1_ref[l])
        for q in range(8):
            t = t + jnp.dot(aggs[l][q], wn1b_ref[l, q],
                            preferred_element_type=jnp.float32)
        upd = jnp.dot(_silu(t), wn2_ref[l], preferred_element_type=jnp.float32)
        h = h + upd + bn2_ref[l]
    o = _silu(jnp.dot(h, wo1_ref[...], preferred_element_type=jnp.float32)
              + bo1_ref[...])
    out_ref[...] = (jnp.dot(o, wo2_ref[...], preferred_element_type=jnp.float32)
                    + bo2_ref[...])


def _node_pipeline(xp, aggrs, Wembp, bembr, Wn1a, Wn1b, bn1r, Wn2, bn2r,
                   Wo1, bo1r, Wo2p, bo2r):
    BN = 1000
    n_b = N_NODES // BN
    full = lambda shape: pl.BlockSpec(shape, lambda n: tuple(0 for _ in shape))
    agspec = pl.BlockSpec((8, BN, 64), lambda n: (0, n, 0))
    return pl.pallas_call(
        _node_body,
        grid=(n_b,),
        in_specs=[
            pl.BlockSpec((BN, 8), lambda n: (n, 0)),
            agspec, agspec, agspec, agspec,
            full((8, HID)),
            full((1, HID)),
            full((NL, HID, HID)),
            full((NL, 8, 64, HID)),
            pl.BlockSpec((NL, 1, HID), lambda n: (0, 0, 0)),
            full((NL, HID, HID)),
            pl.BlockSpec((NL, 1, HID), lambda n: (0, 0, 0)),
            full((HID, HID)),
            full((1, HID)),
            full((HID, 128)),
            full((1, 128)),
        ],
        out_specs=pl.BlockSpec((BN, 128), lambda n: (n, 0)),
        out_shape=jax.ShapeDtypeStruct((N_NODES, 128), jnp.float32),
    )(xp, aggrs[0], aggrs[1], aggrs[2], aggrs[3], Wembp, bembr, Wn1a, Wn1b,
      bn1r, Wn2, bn2r, Wo1, bo1r, Wo2p, bo2r)


def kernel(x, edge_index, edge_attr, W_emb, b_emb, We1, be1, We2, be2,
           Wn1, bn1, Wn2, bn2, Wo1, bo1, Wo2, bo2):
    ei = edge_index.astype(jnp.int32)
    src = ei[0]
    dst = ei[1]
    pad = E_PAD - N_EDGES
    src_p = jnp.pad(src, (0, pad))
    dst_p = jnp.pad(dst, (0, pad))
    ea_p = jnp.pad(edge_attr, ((0, pad), (0, 0)))

    ef = _build_ef(x.reshape(-1), src_p, dst_p, ea_p.reshape(-1))

    We1p = jnp.pad(We1, ((0, 0), (0, 3), (0, 0)))
    be1r = be1.reshape(NL, 1, HID)
    be2r = be2.reshape(NL, 1, HID)
    zeros_hbm = jnp.zeros((128, 64), jnp.float32)
    dst3 = jnp.pad(dst, (0, (N_CHUNKS + 1) * E_CHUNK - N_EDGES)).reshape(
        N_CHUNKS + 1, E_CHUNK)
    aggrs = []
    for l in range(NL):
        msgs = _edge_mlp_layer(ef, We1p[l], be1r[l], We2[l], be2r[l])
        aggrs.append(_segment_sum(msgs, dst3, zeros_hbm))

    xp = jnp.pad(x, ((0, 0), (0, 3)))
    Wembp = jnp.pad(W_emb, ((0, 3), (0, 0)))
    Wo2p = jnp.pad(Wo2, ((0, 0), (0, 125)))
    bo2r = jnp.pad(bo2, (0, 125)).reshape(1, 128)
    Wn1a = Wn1[:, :HID, :]
    Wn1b = Wn1[:, HID:, :].reshape(NL, 8, 64, HID)
    out = _node_pipeline(xp, aggrs, Wembp, b_emb.reshape(1, HID),
                         Wn1a, Wn1b, bn1.reshape(NL, 1, HID), Wn2,
                         bn2.reshape(NL, 1, HID), Wo1, bo1.reshape(1, HID),
                         Wo2p, bo2r)
    return out[:, :3]
